# speedup vs baseline: 11.6522x; 11.6522x over previous
import jax
import jax.numpy as jnp
from jax.experimental import pallas as pl
from jax.experimental.pallas import tpu as pltpu

_SEQ = 16384
_D = 512
_BM = 512
_NB_IN = _SEQ // _BM
_GRID = _NB_IN + 1
_REV = (_SEQ - 1) * 2 + 5


def _upsample_body(x_ref, a_ref, w_ref, out_ref, aux_ref, carry_a, carry_x):
    i = pl.program_id(0)
    lane8 = jax.lax.broadcasted_iota(jnp.int32, (1, 8), 1)

    x = x_ref[...]
    a = a_ref[...]
    lin = jnp.dot(x, w_ref[...], preferred_element_type=jnp.float32)
    A = lin * a[:, 4:5]
    am = a * jnp.where(lane8 < 3, a[:, 3:4], 1.0)

    valid = i < _NB_IN
    A = jnp.where(valid, A, 0.0)
    am = jnp.where(valid, am, 0.0)

    prev_a = jnp.where(i == 0, 0.0, carry_a[6:8, :])
    prev_x = jnp.where(i == 0, 0.0, carry_x[6:8, :])

    ca = jnp.concatenate([prev_a, A], axis=0)
    cx = jnp.concatenate([prev_x, am], axis=0)

    a2, a1, a0 = ca[0:_BM], ca[1:_BM + 1], ca[2:_BM + 2]
    x2, x1, x0 = cx[0:_BM], cx[1:_BM + 1], cx[2:_BM + 2]

    ev = a2 + a1 + a0
    od = a1 + a0
    ev_x = x2 + x1 + x0
    od_x = x1 + x0

    ev = ev / jnp.maximum(ev_x[:, 4:5], 1.0)
    od = od / jnp.maximum(od_x[:, 4:5], 1.0)

    div_e = jnp.where(lane8 < 3, jnp.maximum(ev_x[:, 3:4], 1.0) + 1e-6, 1.0)
    div_o = jnp.where(lane8 < 3, jnp.maximum(od_x[:, 3:4], 1.0) + 1e-6, 1.0)
    ev_x = ev_x / div_e
    od_x = od_x / div_o

    out_ref[...] = jnp.stack([ev, od], axis=1).reshape(2 * _BM, _D)
    aux_ref[...] = jnp.stack([ev_x, od_x], axis=1).reshape(2 * _BM, 8)

    carry_a[...] = A[_BM - 8:_BM, :]
    carry_x[...] = am[_BM - 8:_BM, :]


def kernel(irreps_array, mask_irreps_array, coord, mask_coord, W):
    mc = mask_coord.astype(jnp.float32)[:, None]
    mi = mask_irreps_array.astype(jnp.float32)[:, None]
    aux = jnp.concatenate(
        [coord, mc, mi, jnp.zeros((_SEQ, 3), jnp.float32)], axis=1)

    out, auxout = pl.pallas_call(
        _upsample_body,
        grid=(_GRID,),
        in_specs=[
            pl.BlockSpec((_BM, _D), lambda i: (jnp.minimum(i, _NB_IN - 1), 0)),
            pl.BlockSpec((_BM, 8), lambda i: (jnp.minimum(i, _NB_IN - 1), 0)),
            pl.BlockSpec((_D, _D), lambda i: (0, 0)),
        ],
        out_specs=[
            pl.BlockSpec((2 * _BM, _D), lambda i: (i, 0)),
            pl.BlockSpec((2 * _BM, 8), lambda i: (i, 0)),
        ],
        out_shape=[
            jax.ShapeDtypeStruct((_REV, _D), jnp.float32),
            jax.ShapeDtypeStruct((_REV, 8), jnp.float32),
        ],
        scratch_shapes=[
            pltpu.VMEM((8, _D), jnp.float32),
            pltpu.VMEM((8, 8), jnp.float32),
        ],
        compiler_params=pltpu.CompilerParams(
            dimension_semantics=("arbitrary",)),
    )(irreps_array, aux, W)

    new_coord = auxout[:, 0:3]
    new_mask_coord = auxout[:, 3] > 0.0
    new_mask_irreps = auxout[:, 4] > 0.0
    return out, new_mask_irreps, new_coord, new_mask_coord


# speedup vs baseline: 15.0691x; 1.2932x over previous
import jax
import jax.numpy as jnp
from jax.experimental import pallas as pl
from jax.experimental.pallas import tpu as pltpu

_SEQ = 16384
_D = 512
_BM = 512
_NB_IN = _SEQ // _BM
_GRID = _NB_IN + 1
_REV = (_SEQ - 1) * 2 + 5


def _upsample_body(x_ref, a_ref, w_ref, u_ref, v_ref,
                   out_ref, aux_ref, carry_a, carry_x):
    i = pl.program_id(0)
    lane8 = jax.lax.broadcasted_iota(jnp.int32, (1, 8), 1)

    x = x_ref[...]
    a = a_ref[...]
    lin = jnp.dot(x, w_ref[...], preferred_element_type=jnp.float32)
    A = lin * a[:, 4:5]
    am = a * jnp.where(lane8 < 3, a[:, 3:4], 1.0)

    valid = i < _NB_IN
    A = jnp.where(valid, A, 0.0)
    am = jnp.where(valid, am, 0.0)

    pa = jnp.where(i == 0, 0.0, carry_a[...])
    px = jnp.where(i == 0, 0.0, carry_x[...])

    u = u_ref[...]
    v = v_ref[...]
    out_raw = (jnp.dot(u, A, preferred_element_type=jnp.float32)
               + jnp.dot(v, pa, preferred_element_type=jnp.float32))
    aux_raw = (jnp.dot(u, am, preferred_element_type=jnp.float32)
               + jnp.dot(v, px, preferred_element_type=jnp.float32))

    out_ref[...] = out_raw / jnp.maximum(aux_raw[:, 4:5], 1.0)
    div = jnp.where(lane8 < 3, jnp.maximum(aux_raw[:, 3:4], 1.0) + 1e-6, 1.0)
    aux_ref[...] = aux_raw / div

    carry_a[...] = A[_BM - 8:_BM, :]
    carry_x[...] = am[_BM - 8:_BM, :]


def kernel(irreps_array, mask_irreps_array, coord, mask_coord, W):
    mc = mask_coord.astype(jnp.float32)[:, None]
    mi = mask_irreps_array.astype(jnp.float32)[:, None]
    aux = jnp.concatenate(
        [coord, mc, mi, jnp.zeros((_SEQ, 3), jnp.float32)], axis=1)

    r_idx = jnp.arange(2 * _BM)[:, None]
    c_idx = jnp.arange(_BM)[None, :]
    t = r_idx - 2 * c_idx
    u_mat = ((t >= 0) & (t <= 4)).astype(jnp.float32)
    c8 = jnp.arange(8)[None, :]
    tv = r_idx + 16 - 2 * c8
    v_mat = ((tv >= 0) & (tv <= 4)).astype(jnp.float32)

    out, auxout = pl.pallas_call(
        _upsample_body,
        grid=(_GRID,),
        in_specs=[
            pl.BlockSpec((_BM, _D), lambda i: (jnp.minimum(i, _NB_IN - 1), 0)),
            pl.BlockSpec((_BM, 8), lambda i: (jnp.minimum(i, _NB_IN - 1), 0)),
            pl.BlockSpec((_D, _D), lambda i: (0, 0)),
            pl.BlockSpec((2 * _BM, _BM), lambda i: (0, 0)),
            pl.BlockSpec((2 * _BM, 8), lambda i: (0, 0)),
        ],
        out_specs=[
            pl.BlockSpec((2 * _BM, _D), lambda i: (i, 0)),
            pl.BlockSpec((2 * _BM, 8), lambda i: (i, 0)),
        ],
        out_shape=[
            jax.ShapeDtypeStruct((_REV, _D), jnp.float32),
            jax.ShapeDtypeStruct((_REV, 8), jnp.float32),
        ],
        scratch_shapes=[
            pltpu.VMEM((8, _D), jnp.float32),
            pltpu.VMEM((8, 8), jnp.float32),
        ],
        compiler_params=pltpu.CompilerParams(
            dimension_semantics=("arbitrary",)),
    )(irreps_array, aux, W, u_mat, v_mat)

    new_coord = auxout[:, 0:3]
    new_mask_coord = auxout[:, 3] > 0.0
    new_mask_irreps = auxout[:, 4] > 0.0
    return out, new_mask_irreps, new_coord, new_mask_coord


# speedup vs baseline: 16.6020x; 1.1017x over previous
import jax
import jax.numpy as jnp
from jax.experimental import pallas as pl
from jax.experimental.pallas import tpu as pltpu

_SEQ = 16384
_D = 512
_BM = 512
_NB_IN = _SEQ // _BM
_GRID = _NB_IN + 1
_REV = (_SEQ - 1) * 2 + 5


_SB = 128
_NSB = _BM // _SB


def _upsample_body(x_ref, a_ref, w_ref, u_ref, v_ref,
                   out_ref, aux_ref, carry_a, carry_x):
    i = pl.program_id(0)
    lane8 = jax.lax.broadcasted_iota(jnp.int32, (1, 8), 1)

    x = x_ref[...]
    a = a_ref[...]
    lin = jnp.dot(x, w_ref[...], preferred_element_type=jnp.float32)
    A = lin * a[:, 4:5]
    am = a * jnp.where(lane8 < 3, a[:, 3:4], 1.0)

    valid = i < _NB_IN
    A = jnp.where(valid, A, 0.0)
    am = jnp.where(valid, am, 0.0)

    u = u_ref[...]
    v = v_ref[...]
    for k in range(_NSB):
        asub = A[k * _SB:(k + 1) * _SB]
        xsub = am[k * _SB:(k + 1) * _SB]
        if k == 0:
            pa = jnp.where(i == 0, 0.0, carry_a[...])
            px = jnp.where(i == 0, 0.0, carry_x[...])
        else:
            pa = A[k * _SB - 8:k * _SB]
            px = am[k * _SB - 8:k * _SB]
        out_raw = (jnp.dot(u, asub, preferred_element_type=jnp.float32)
                   + jnp.dot(v, pa, preferred_element_type=jnp.float32))
        aux_raw = (jnp.dot(u, xsub, preferred_element_type=jnp.float32)
                   + jnp.dot(v, px, preferred_element_type=jnp.float32))

        sl = slice(2 * _SB * k, 2 * _SB * (k + 1))
        out_ref[sl, :] = out_raw / jnp.maximum(aux_raw[:, 4:5], 1.0)
        div = jnp.where(lane8 < 3,
                        jnp.maximum(aux_raw[:, 3:4], 1.0) + 1e-6, 1.0)
        aux_ref[sl, :] = aux_raw / div

    carry_a[...] = A[_BM - 8:_BM, :]
    carry_x[...] = am[_BM - 8:_BM, :]


def kernel(irreps_array, mask_irreps_array, coord, mask_coord, W):
    mc = mask_coord.astype(jnp.float32)[:, None]
    mi = mask_irreps_array.astype(jnp.float32)[:, None]
    aux = jnp.concatenate(
        [coord, mc, mi, jnp.zeros((_SEQ, 3), jnp.float32)], axis=1)

    r_idx = jnp.arange(2 * _SB)[:, None]
    c_idx = jnp.arange(_SB)[None, :]
    t = r_idx - 2 * c_idx
    u_mat = ((t >= 0) & (t <= 4)).astype(jnp.float32)
    c8 = jnp.arange(8)[None, :]
    tv = r_idx + 16 - 2 * c8
    v_mat = ((tv >= 0) & (tv <= 4)).astype(jnp.float32)

    out, auxout = pl.pallas_call(
        _upsample_body,
        grid=(_GRID,),
        in_specs=[
            pl.BlockSpec((_BM, _D), lambda i: (jnp.minimum(i, _NB_IN - 1), 0)),
            pl.BlockSpec((_BM, 8), lambda i: (jnp.minimum(i, _NB_IN - 1), 0)),
            pl.BlockSpec((_D, _D), lambda i: (0, 0)),
            pl.BlockSpec((2 * _SB, _SB), lambda i: (0, 0)),
            pl.BlockSpec((2 * _SB, 8), lambda i: (0, 0)),
        ],
        out_specs=[
            pl.BlockSpec((2 * _BM, _D), lambda i: (i, 0)),
            pl.BlockSpec((2 * _BM, 8), lambda i: (i, 0)),
        ],
        out_shape=[
            jax.ShapeDtypeStruct((_REV, _D), jnp.float32),
            jax.ShapeDtypeStruct((_REV, 8), jnp.float32),
        ],
        scratch_shapes=[
            pltpu.VMEM((8, _D), jnp.float32),
            pltpu.VMEM((8, 8), jnp.float32),
        ],
        compiler_params=pltpu.CompilerParams(
            dimension_semantics=("arbitrary",)),
    )(irreps_array, aux, W, u_mat, v_mat)

    new_coord = auxout[:, 0:3]
    new_mask_coord = auxout[:, 3] > 0.0
    new_mask_irreps = auxout[:, 4] > 0.0
    return out, new_mask_irreps, new_coord, new_mask_coord


# speedup vs baseline: 18.3687x; 1.1064x over previous
import jax
import jax.numpy as jnp
from jax.experimental import pallas as pl
from jax.experimental.pallas import tpu as pltpu

_SEQ = 16384
_D = 512
_BM = 1024
_NB_IN = _SEQ // _BM
_GRID = _NB_IN + 1
_REV = (_SEQ - 1) * 2 + 5


_SB = 128
_NSB = _BM // _SB


def _upsample_body(x_ref, a_ref, w_ref, u_ref, v_ref,
                   out_ref, aux_ref, carry_a, carry_x):
    i = pl.program_id(0)
    lane8 = jax.lax.broadcasted_iota(jnp.int32, (1, 8), 1)

    x = x_ref[...]
    a = a_ref[...]
    lin = jnp.dot(x, w_ref[...], preferred_element_type=jnp.float32)
    A = lin * a[:, 4:5]
    am = a * jnp.where(lane8 < 3, a[:, 3:4], 1.0)

    valid = i < _NB_IN
    A = jnp.where(valid, A, 0.0)
    am = jnp.where(valid, am, 0.0)

    u = u_ref[...]
    v = v_ref[...]
    for k in range(_NSB):
        asub = A[k * _SB:(k + 1) * _SB]
        xsub = am[k * _SB:(k + 1) * _SB]
        if k == 0:
            pa = jnp.where(i == 0, 0.0, carry_a[...])
            px = jnp.where(i == 0, 0.0, carry_x[...])
        else:
            pa = A[k * _SB - 8:k * _SB]
            px = am[k * _SB - 8:k * _SB]
        out_raw = (jnp.dot(u, asub, preferred_element_type=jnp.float32)
                   + jnp.dot(v, pa, preferred_element_type=jnp.float32))
        aux_raw = (jnp.dot(u, xsub, preferred_element_type=jnp.float32)
                   + jnp.dot(v, px, preferred_element_type=jnp.float32))

        sl = slice(2 * _SB * k, 2 * _SB * (k + 1))
        out_ref[sl, :] = out_raw / jnp.maximum(aux_raw[:, 4:5], 1.0)
        div = jnp.where(lane8 < 3,
                        jnp.maximum(aux_raw[:, 3:4], 1.0) + 1e-6, 1.0)
        aux_ref[sl, :] = aux_raw / div

    carry_a[...] = A[_BM - 8:_BM, :]
    carry_x[...] = am[_BM - 8:_BM, :]


def kernel(irreps_array, mask_irreps_array, coord, mask_coord, W):
    mc = mask_coord.astype(jnp.float32)[:, None]
    mi = mask_irreps_array.astype(jnp.float32)[:, None]
    aux = jnp.concatenate(
        [coord, mc, mi, jnp.zeros((_SEQ, 3), jnp.float32)], axis=1)

    r_idx = jnp.arange(2 * _SB)[:, None]
    c_idx = jnp.arange(_SB)[None, :]
    t = r_idx - 2 * c_idx
    u_mat = ((t >= 0) & (t <= 4)).astype(jnp.float32)
    c8 = jnp.arange(8)[None, :]
    tv = r_idx + 16 - 2 * c8
    v_mat = ((tv >= 0) & (tv <= 4)).astype(jnp.float32)

    out, auxout = pl.pallas_call(
        _upsample_body,
        grid=(_GRID,),
        in_specs=[
            pl.BlockSpec((_BM, _D), lambda i: (jnp.minimum(i, _NB_IN - 1), 0)),
            pl.BlockSpec((_BM, 8), lambda i: (jnp.minimum(i, _NB_IN - 1), 0)),
            pl.BlockSpec((_D, _D), lambda i: (0, 0)),
            pl.BlockSpec((2 * _SB, _SB), lambda i: (0, 0)),
            pl.BlockSpec((2 * _SB, 8), lambda i: (0, 0)),
        ],
        out_specs=[
            pl.BlockSpec((2 * _BM, _D), lambda i: (i, 0)),
            pl.BlockSpec((2 * _BM, 8), lambda i: (i, 0)),
        ],
        out_shape=[
            jax.ShapeDtypeStruct((_REV, _D), jnp.float32),
            jax.ShapeDtypeStruct((_REV, 8), jnp.float32),
        ],
        scratch_shapes=[
            pltpu.VMEM((8, _D), jnp.float32),
            pltpu.VMEM((8, 8), jnp.float32),
        ],
        compiler_params=pltpu.CompilerParams(
            dimension_semantics=("arbitrary",)),
    )(irreps_array, aux, W, u_mat, v_mat)

    new_coord = auxout[:, 0:3]
    new_mask_coord = auxout[:, 3] > 0.0
    new_mask_irreps = auxout[:, 4] > 0.0
    return out, new_mask_irreps, new_coord, new_mask_coord


# speedup vs baseline: 19.1793x; 1.0441x over previous
import jax
import jax.numpy as jnp
from jax.experimental import pallas as pl
from jax.experimental.pallas import tpu as pltpu

_SEQ = 16384
_D = 512
_BM = 2048
_NB_IN = _SEQ // _BM
_GRID = _NB_IN + 1
_REV = (_SEQ - 1) * 2 + 5


_SB = 128
_NSB = _BM // _SB


def _upsample_body(x_ref, a_ref, w_ref, u_ref, v_ref,
                   out_ref, aux_ref, carry_a, carry_x):
    i = pl.program_id(0)
    lane8 = jax.lax.broadcasted_iota(jnp.int32, (1, 8), 1)

    x = x_ref[...]
    a = a_ref[...]
    lin = jnp.dot(x, w_ref[...], preferred_element_type=jnp.float32)
    A = lin * a[:, 4:5]
    am = a * jnp.where(lane8 < 3, a[:, 3:4], 1.0)

    valid = i < _NB_IN
    A = jnp.where(valid, A, 0.0)
    am = jnp.where(valid, am, 0.0)

    u = u_ref[...]
    v = v_ref[...]
    for k in range(_NSB):
        asub = A[k * _SB:(k + 1) * _SB]
        xsub = am[k * _SB:(k + 1) * _SB]
        if k == 0:
            pa = jnp.where(i == 0, 0.0, carry_a[...])
            px = jnp.where(i == 0, 0.0, carry_x[...])
        else:
            pa = A[k * _SB - 8:k * _SB]
            px = am[k * _SB - 8:k * _SB]
        out_raw = (jnp.dot(u, asub, preferred_element_type=jnp.float32)
                   + jnp.dot(v, pa, preferred_element_type=jnp.float32))
        aux_raw = (jnp.dot(u, xsub, preferred_element_type=jnp.float32)
                   + jnp.dot(v, px, preferred_element_type=jnp.float32))

        sl = slice(2 * _SB * k, 2 * _SB * (k + 1))
        out_ref[sl, :] = out_raw / jnp.maximum(aux_raw[:, 4:5], 1.0)
        div = jnp.where(lane8 < 3,
                        jnp.maximum(aux_raw[:, 3:4], 1.0) + 1e-6, 1.0)
        aux_ref[sl, :] = aux_raw / div

    carry_a[...] = A[_BM - 8:_BM, :]
    carry_x[...] = am[_BM - 8:_BM, :]


def kernel(irreps_array, mask_irreps_array, coord, mask_coord, W):
    mc = mask_coord.astype(jnp.float32)[:, None]
    mi = mask_irreps_array.astype(jnp.float32)[:, None]
    aux = jnp.concatenate(
        [coord, mc, mi, jnp.zeros((_SEQ, 3), jnp.float32)], axis=1)

    r_idx = jnp.arange(2 * _SB)[:, None]
    c_idx = jnp.arange(_SB)[None, :]
    t = r_idx - 2 * c_idx
    u_mat = ((t >= 0) & (t <= 4)).astype(jnp.float32)
    c8 = jnp.arange(8)[None, :]
    tv = r_idx + 16 - 2 * c8
    v_mat = ((tv >= 0) & (tv <= 4)).astype(jnp.float32)

    out, auxout = pl.pallas_call(
        _upsample_body,
        grid=(_GRID,),
        in_specs=[
            pl.BlockSpec((_BM, _D), lambda i: (jnp.minimum(i, _NB_IN - 1), 0)),
            pl.BlockSpec((_BM, 8), lambda i: (jnp.minimum(i, _NB_IN - 1), 0)),
            pl.BlockSpec((_D, _D), lambda i: (0, 0)),
            pl.BlockSpec((2 * _SB, _SB), lambda i: (0, 0)),
            pl.BlockSpec((2 * _SB, 8), lambda i: (0, 0)),
        ],
        out_specs=[
            pl.BlockSpec((2 * _BM, _D), lambda i: (i, 0)),
            pl.BlockSpec((2 * _BM, 8), lambda i: (i, 0)),
        ],
        out_shape=[
            jax.ShapeDtypeStruct((_REV, _D), jnp.float32),
            jax.ShapeDtypeStruct((_REV, 8), jnp.float32),
        ],
        scratch_shapes=[
            pltpu.VMEM((8, _D), jnp.float32),
            pltpu.VMEM((8, 8), jnp.float32),
        ],
        compiler_params=pltpu.CompilerParams(
            dimension_semantics=("arbitrary",)),
    )(irreps_array, aux, W, u_mat, v_mat)

    new_coord = auxout[:, 0:3]
    new_mask_coord = auxout[:, 3] > 0.0
    new_mask_irreps = auxout[:, 4] > 0.0
    return out, new_mask_irreps, new_coord, new_mask_coord
